# baseline (speedup 1.0000x reference)
import functools

import jax
import jax.numpy as jnp
from jax import lax
from jax.experimental import pallas as pl
from jax.experimental.pallas import tpu as pltpu
from jax.experimental.pallas import tpu_sc as plsc

B = 16384
E = 16
T = 20
NC = 2
NS = 16
NW = NC * NS
PERW = B // NW
CB = 128
CA = 128
NCHA = PERW // CA
CT = CA * T
NG = CT // 128

_EPS = 1e-8
_EPS2 = 1e-16

_MESH = plsc.VectorSubcoreMesh(core_axis_name="c", subcore_axis_name="s",
                               num_cores=NC, num_subcores=NS)


def _rsqrt(x):
    i = lax.bitcast_convert_type(x, jnp.int32)
    i = jnp.int32(0x5F3759DF) - lax.shift_right_logical(i, 1)
    y = lax.bitcast_convert_type(i, jnp.float32)
    for _ in range(3):
        y = y * (1.5 - 0.5 * x * y * y)
    return y


def _wid():
    return lax.axis_index("s") * NC + lax.axis_index("c")


def _content_body(uti_hbm, w_hbm, iti_hbm, tt_hbm, out_hbm,
                  utidx_v, itidx_v, w_v, utag_rows, itag_rows, uc_buf,
                  ic_buf, out_v, sem_tag):
    base0 = _wid() * PERW
    lanes = lax.iota(jnp.int32, 16)

    def chunk_body(ci, carry):
        base = pl.multiple_of(base0 + ci * CA, CA)

        pltpu.sync_copy(uti_hbm.at[pl.ds(base, CA), :], utidx_v)
        pltpu.sync_copy(iti_hbm.at[pl.ds(base, CA), :], itidx_v)
        pltpu.sync_copy(w_hbm.at[pl.ds(base, CA), :], w_v)

        def fire(r, c):
            pltpu.async_copy(tt_hbm.at[utidx_v.at[r, :]], utag_rows.at[r],
                             sem_tag)
            pltpu.async_copy(tt_hbm.at[itidx_v.at[r, :]], itag_rows.at[r],
                             sem_tag)
            return c

        lax.fori_loop(0, CA, fire, 0)

        def draint(r, c):
            pltpu.make_async_copy(tt_hbm.at[utidx_v.at[r, :]],
                                  utag_rows.at[r], sem_tag).wait()
            pltpu.make_async_copy(tt_hbm.at[itidx_v.at[r, :]],
                                  itag_rows.at[r], sem_tag).wait()
            return c

        lax.fori_loop(0, CA, draint, 0)

        def elem(b, c):
            bv = jnp.broadcast_to(b, (16,))
            w0 = plsc.load_gather(w_v, [bv, jnp.full((16,), 0, jnp.int32)])
            w1 = plsc.load_gather(w_v, [bv, jnp.full((16,), 1, jnp.int32)])
            uc0 = utag_rows[b, 0, :] * w0
            uc1 = utag_rows[b, 1, :] * w1
            ic0 = itag_rows[b, 0, :]
            ic1 = itag_rows[b, 1, :]
            for t in range(2, T, 2):
                wt0 = plsc.load_gather(w_v,
                                       [bv, jnp.full((16,), t, jnp.int32)])
                wt1 = plsc.load_gather(w_v,
                                       [bv, jnp.full((16,), t + 1, jnp.int32)])
                uc0 = uc0 + utag_rows[b, t, :] * wt0
                uc1 = uc1 + utag_rows[b, t + 1, :] * wt1
                ic0 = ic0 + itag_rows[b, t, :]
                ic1 = ic1 + itag_rows[b, t + 1, :]
            uc_buf[b, :] = uc0 + uc1
            ic_buf[b, :] = ic0 + ic1
            return c

        lax.fori_loop(0, CA, elem, 0)

        def group(g, c):
            rows = g * 16 + lanes
            zero = jnp.zeros((16,), jnp.float32)
            dotk = zero
            sa = zero
            sb = zero
            wsum = zero
            for e in range(E):
                ce = jnp.full((16,), e, jnp.int32)
                ae = plsc.load_gather(uc_buf, [rows, ce])
                be = plsc.load_gather(ic_buf, [rows, ce])
                dotk = dotk + ae * be
                sa = sa + ae * ae
                sb = sb + be * be
            for t in range(T):
                wsum = wsum + plsc.load_gather(
                    w_v, [rows, jnp.full((16,), t, jnp.int32)])

            s_u = 1.0 / (wsum + _EPS)
            na2 = jnp.maximum(sa * s_u * s_u, _EPS2)
            nb2 = jnp.maximum(sb * (1.0 / (T * T)), _EPS2)
            content = dotk * (s_u * (1.0 / T)) * _rsqrt(na2 * nb2)
            off = pl.multiple_of(g * 16, 16)
            out_v[pl.ds(off, 16)] = content
            return c

        lax.fori_loop(0, CA // 16, group, 0)
        pltpu.sync_copy(out_v, out_hbm.at[pl.ds(base, CA)])
        return carry

    lax.fori_loop(0, NCHA, chunk_body, 0)


_content_call = functools.partial(
    pl.kernel,
    out_type=jax.ShapeDtypeStruct((B,), jnp.float32),
    mesh=_MESH,
    compiler_params=pltpu.CompilerParams(needs_layout_passes=False,
                                         use_tc_tiling_on_sc=False),
    scratch_types=[
        pltpu.VMEM((CA, T), jnp.int32),
        pltpu.VMEM((CA, T), jnp.int32),
        pltpu.VMEM((CA, T), jnp.float32),
        pltpu.VMEM((CA, T, E), jnp.float32),
        pltpu.VMEM((CA, T, E), jnp.float32),
        pltpu.VMEM((CA, E), jnp.float32),
        pltpu.VMEM((CA, E), jnp.float32),
        pltpu.VMEM((CA,), jnp.float32),
        pltpu.SemaphoreType.DMA,
    ],
)(_content_body)


def _collab_body(ui_hbm, ii_hbm, content_hbm, ut_hbm, it_hbm, out_hbm,
                 uidx_v, iidx_v, urows_t, irows_t, content_v, out_v, sem):
    base0 = pl.multiple_of(_wid() * PERW, PERW)
    lanes = lax.iota(jnp.int32, 16)

    pltpu.sync_copy(ui_hbm.at[pl.ds(base0, PERW)], uidx_v)
    pltpu.sync_copy(ii_hbm.at[pl.ds(base0, PERW)], iidx_v)
    pltpu.sync_copy(content_hbm.at[pl.ds(base0, PERW)], content_v)

    def chunk(ci, carry):
        cbase = pl.multiple_of(ci * CB, CB)

        def fire(g, c):
            off = pl.multiple_of(cbase + g * 16, 16)
            uv = uidx_v[pl.ds(off, 16)]
            iv = iidx_v[pl.ds(off, 16)]
            for t in range(16):
                pltpu.async_copy(ut_hbm.at[pl.ds(uv[t], 1), :],
                                 urows_t.at[pl.ds(g * 16 + t, 1), :], sem)
                pltpu.async_copy(it_hbm.at[pl.ds(iv[t], 1), :],
                                 irows_t.at[pl.ds(g * 16 + t, 1), :], sem)
            return c

        lax.fori_loop(0, CB // 16, fire, 0)

        def drain(g, c):
            off = pl.multiple_of(cbase + g * 16, 16)
            uv = uidx_v[pl.ds(off, 16)]
            iv = iidx_v[pl.ds(off, 16)]
            for t in range(16):
                pltpu.make_async_copy(ut_hbm.at[pl.ds(uv[t], 1), :],
                                      urows_t.at[pl.ds(g * 16 + t, 1), :],
                                      sem).wait()
                pltpu.make_async_copy(it_hbm.at[pl.ds(iv[t], 1), :],
                                      irows_t.at[pl.ds(g * 16 + t, 1), :],
                                      sem).wait()
            return c

        lax.fori_loop(0, CB // 16, drain, 0)

        def group(g, c):
            rows = g * 16 + lanes
            zero = jnp.zeros((16,), jnp.float32)
            dotc = zero
            su = zero
            sv = zero
            for e in range(E):
                ce = jnp.full((16,), e, jnp.int32)
                ue = plsc.load_gather(urows_t, [rows, ce])
                ve = plsc.load_gather(irows_t, [rows, ce])
                dotc = dotc + ue * ve
                su = su + ue * ue
                sv = sv + ve * ve
            collab = dotc * _rsqrt(jnp.maximum(su, _EPS2) *
                                   jnp.maximum(sv, _EPS2))
            off = pl.multiple_of(cbase + g * 16, 16)
            content = content_v[pl.ds(off, 16)]
            out_v[pl.ds(off, 16)] = 0.5 * collab + 0.5 * content
            return c

        lax.fori_loop(0, CB // 16, group, 0)
        return carry

    lax.fori_loop(0, PERW // CB, chunk, 0)
    pltpu.sync_copy(out_v, out_hbm.at[pl.ds(base0, PERW)])


_collab_call = functools.partial(
    pl.kernel,
    out_type=jax.ShapeDtypeStruct((B,), jnp.float32),
    mesh=_MESH,
    compiler_params=pltpu.CompilerParams(needs_layout_passes=False,
                                         use_tc_tiling_on_sc=True,
                                         has_side_effects=False),
    scratch_types=[
        pltpu.VMEM((PERW,), jnp.int32),
        pltpu.VMEM((PERW,), jnp.int32),
        pltpu.VMEM((CB, E), jnp.float32),
        pltpu.VMEM((CB, E), jnp.float32),
        pltpu.VMEM((PERW,), jnp.float32),
        pltpu.VMEM((PERW,), jnp.float32),
        pltpu.SemaphoreType.DMA,
    ],
)(_collab_body)


def kernel(user_idx, item_idx, user_tag_idx, user_tag_weights, item_tag_idx,
           user_table, item_table, tag_table):
    ui = user_idx.astype(jnp.int32)
    ii = item_idx.astype(jnp.int32)
    uti = user_tag_idx.astype(jnp.int32)
    w = user_tag_weights.astype(jnp.float32)
    iti = item_tag_idx.astype(jnp.int32)
    content = _content_call(uti, w, iti, tag_table)
    return _collab_call(ui, ii, content, user_table, item_table)


# speedup vs baseline: 1.0250x; 1.0250x over previous
import functools

import jax
import jax.numpy as jnp
from jax import lax
from jax.experimental import pallas as pl
from jax.experimental.pallas import tpu as pltpu
from jax.experimental.pallas import tpu_sc as plsc

B = 16384
E = 16
T = 20
NC = 2
NS = 16
NW = NC * NS
PERW = B // NW
CB = 128
CA = 128
NCHA = PERW // CA
CT = CA * T
NG = CT // 128

_EPS = 1e-8
_EPS2 = 1e-16

_MESH = plsc.VectorSubcoreMesh(core_axis_name="c", subcore_axis_name="s",
                               num_cores=NC, num_subcores=NS)


def _rsqrt(x):
    i = lax.bitcast_convert_type(x, jnp.int32)
    i = jnp.int32(0x5F3759DF) - lax.shift_right_logical(i, 1)
    y = lax.bitcast_convert_type(i, jnp.float32)
    for _ in range(3):
        y = y * (1.5 - 0.5 * x * y * y)
    return y


def _wid():
    return lax.axis_index("s") * NC + lax.axis_index("c")


def _content_body(uti_hbm, w_hbm, iti_hbm, tt_hbm, out_hbm,
                  tidx_v, w_v, tag_rows, uc_buf, ic_buf, out_v, sem_tag):
    base0 = _wid() * PERW
    lanes = lax.iota(jnp.int32, 16)

    def chunk_body(ci, carry):
        base = pl.multiple_of(base0 + ci * CA, CA)
        tbase = pl.multiple_of(base * T, CT)

        pltpu.sync_copy(uti_hbm.at[pl.ds(tbase, CT)], tidx_v.at[pl.ds(0, CT)])
        pltpu.sync_copy(iti_hbm.at[pl.ds(tbase, CT)], tidx_v.at[pl.ds(CT, CT)])
        pltpu.sync_copy(w_hbm.at[pl.ds(tbase, CT)], w_v)

        def fire(k, c):
            off = pl.multiple_of(k * 128, 128)
            pltpu.async_copy(tt_hbm.at[tidx_v.at[pl.ds(off, 128)]],
                             tag_rows.at[pl.ds(off, 128)], sem_tag)
            return c

        lax.fori_loop(0, 2 * NG, fire, 0)
        pltpu.make_async_copy(tt_hbm.at[pl.ds(0, 2 * CT)], tag_rows,
                              sem_tag).wait()

        def elem(b, c):
            j = b * T
            jv = jnp.broadcast_to(j, (16,))
            w0 = plsc.load_gather(w_v, [jv])
            w1 = plsc.load_gather(w_v, [jv + 1])
            uc0 = tag_rows[j, :] * w0
            uc1 = tag_rows[j + 1, :] * w1
            ic0 = tag_rows[CT + j, :]
            ic1 = tag_rows[CT + j + 1, :]
            for t in range(2, T, 2):
                wt0 = plsc.load_gather(w_v, [jv + t])
                wt1 = plsc.load_gather(w_v, [jv + t + 1])
                uc0 = uc0 + tag_rows[j + t, :] * wt0
                uc1 = uc1 + tag_rows[j + t + 1, :] * wt1
                ic0 = ic0 + tag_rows[CT + j + t, :]
                ic1 = ic1 + tag_rows[CT + j + t + 1, :]
            uc_buf[b, :] = uc0 + uc1
            ic_buf[b, :] = ic0 + ic1
            return c

        lax.fori_loop(0, CA, elem, 0)

        def group(g, c):
            rows = g * 16 + lanes
            rows20 = rows * T
            zero = jnp.zeros((16,), jnp.float32)
            dotk = zero
            sa = zero
            sb = zero
            wsum = zero
            for e in range(E):
                ce = jnp.full((16,), e, jnp.int32)
                ae = plsc.load_gather(uc_buf, [rows, ce])
                be = plsc.load_gather(ic_buf, [rows, ce])
                dotk = dotk + ae * be
                sa = sa + ae * ae
                sb = sb + be * be
            for t in range(T):
                wsum = wsum + plsc.load_gather(w_v, [rows20 + t])

            s_u = 1.0 / (wsum + _EPS)
            na2 = jnp.maximum(sa * s_u * s_u, _EPS2)
            nb2 = jnp.maximum(sb * (1.0 / (T * T)), _EPS2)
            content = dotk * (s_u * (1.0 / T)) * _rsqrt(na2 * nb2)
            off = pl.multiple_of(g * 16, 16)
            out_v[pl.ds(off, 16)] = content
            return c

        lax.fori_loop(0, CA // 16, group, 0)
        pltpu.sync_copy(out_v, out_hbm.at[pl.ds(base, CA)])
        return carry

    lax.fori_loop(0, NCHA, chunk_body, 0)


_content_call = functools.partial(
    pl.kernel,
    out_type=jax.ShapeDtypeStruct((B,), jnp.float32),
    mesh=_MESH,
    compiler_params=pltpu.CompilerParams(needs_layout_passes=False,
                                         use_tc_tiling_on_sc=False),
    scratch_types=[
        pltpu.VMEM((2 * CT,), jnp.int32),
        pltpu.VMEM((CT,), jnp.float32),
        pltpu.VMEM((2 * CT, E), jnp.float32),
        pltpu.VMEM((CA, E), jnp.float32),
        pltpu.VMEM((CA, E), jnp.float32),
        pltpu.VMEM((CA,), jnp.float32),
        pltpu.SemaphoreType.DMA,
    ],
)(_content_body)


def _collab_body(ui_hbm, ii_hbm, content_hbm, ut_hbm, it_hbm, out_hbm,
                 uidx_v, iidx_v, urows_t, irows_t, content_v, out_v, sem):
    base0 = pl.multiple_of(_wid() * PERW, PERW)
    lanes = lax.iota(jnp.int32, 16)

    pltpu.sync_copy(ui_hbm.at[pl.ds(base0, PERW)], uidx_v)
    pltpu.sync_copy(ii_hbm.at[pl.ds(base0, PERW)], iidx_v)
    pltpu.sync_copy(content_hbm.at[pl.ds(base0, PERW)], content_v)

    def chunk(ci, carry):
        cbase = pl.multiple_of(ci * CB, CB)

        def fire(g, c):
            off = pl.multiple_of(cbase + g * 16, 16)
            uv = uidx_v[pl.ds(off, 16)]
            iv = iidx_v[pl.ds(off, 16)]
            for t in range(16):
                pltpu.async_copy(ut_hbm.at[pl.ds(uv[t], 1), :],
                                 urows_t.at[pl.ds(g * 16 + t, 1), :], sem)
                pltpu.async_copy(it_hbm.at[pl.ds(iv[t], 1), :],
                                 irows_t.at[pl.ds(g * 16 + t, 1), :], sem)
            return c

        lax.fori_loop(0, CB // 16, fire, 0)

        def drain(g, c):
            off = pl.multiple_of(cbase + g * 16, 16)
            uv = uidx_v[pl.ds(off, 16)]
            iv = iidx_v[pl.ds(off, 16)]
            for t in range(16):
                pltpu.make_async_copy(ut_hbm.at[pl.ds(uv[t], 1), :],
                                      urows_t.at[pl.ds(g * 16 + t, 1), :],
                                      sem).wait()
                pltpu.make_async_copy(it_hbm.at[pl.ds(iv[t], 1), :],
                                      irows_t.at[pl.ds(g * 16 + t, 1), :],
                                      sem).wait()
            return c

        lax.fori_loop(0, CB // 16, drain, 0)

        def group(g, c):
            rows = g * 16 + lanes
            zero = jnp.zeros((16,), jnp.float32)
            dotc = zero
            su = zero
            sv = zero
            for e in range(E):
                ce = jnp.full((16,), e, jnp.int32)
                ue = plsc.load_gather(urows_t, [rows, ce])
                ve = plsc.load_gather(irows_t, [rows, ce])
                dotc = dotc + ue * ve
                su = su + ue * ue
                sv = sv + ve * ve
            collab = dotc * _rsqrt(jnp.maximum(su, _EPS2) *
                                   jnp.maximum(sv, _EPS2))
            off = pl.multiple_of(cbase + g * 16, 16)
            content = content_v[pl.ds(off, 16)]
            out_v[pl.ds(off, 16)] = 0.5 * collab + 0.5 * content
            return c

        lax.fori_loop(0, CB // 16, group, 0)
        return carry

    lax.fori_loop(0, PERW // CB, chunk, 0)
    pltpu.sync_copy(out_v, out_hbm.at[pl.ds(base0, PERW)])


_collab_call = functools.partial(
    pl.kernel,
    out_type=jax.ShapeDtypeStruct((B,), jnp.float32),
    mesh=_MESH,
    compiler_params=pltpu.CompilerParams(needs_layout_passes=False,
                                         use_tc_tiling_on_sc=True,
                                         has_side_effects=False),
    scratch_types=[
        pltpu.VMEM((PERW,), jnp.int32),
        pltpu.VMEM((PERW,), jnp.int32),
        pltpu.VMEM((CB, E), jnp.float32),
        pltpu.VMEM((CB, E), jnp.float32),
        pltpu.VMEM((PERW,), jnp.float32),
        pltpu.VMEM((PERW,), jnp.float32),
        pltpu.SemaphoreType.DMA,
    ],
)(_collab_body)


def kernel(user_idx, item_idx, user_tag_idx, user_tag_weights, item_tag_idx,
           user_table, item_table, tag_table):
    ui = user_idx.astype(jnp.int32)
    ii = item_idx.astype(jnp.int32)
    uti = user_tag_idx.reshape(-1).astype(jnp.int32)
    w = user_tag_weights.reshape(-1).astype(jnp.float32)
    iti = item_tag_idx.reshape(-1).astype(jnp.int32)
    content = _content_call(uti, w, iti, tag_table)
    return _collab_call(ui, ii, content, user_table, item_table)
